# baseline (speedup 1.0000x reference)
import dataclasses
import functools
import math

import jax
import jax.numpy as jnp
from jax import lax
from jax.experimental import pallas as pl
from jax.experimental.pallas import tpu as pltpu
from jax.experimental.pallas import tpu_sc as plsc

_SIZE = 100000
_PAD = 0
_SMOOTHING = 0.1
_CONF = 1.0 - _SMOOTHING
_EPS = _SMOOTHING / (_SIZE - 2)
_C = (_SIZE - 2) * _EPS * math.log(_EPS) + _CONF * math.log(_CONF)

_N = 1024
_CB = 4096
_WAVE = 64


def _sc_gather(target, x):
    mesh = plsc.ScalarSubcoreMesh(axis_name="c", num_cores=2)
    half = _N // 2

    @functools.partial(
        pl.kernel,
        out_type=jax.ShapeDtypeStruct((_N,), jnp.float32),
        mesh=mesh,
        scratch_types=[
            pltpu.SMEM((half,), jnp.int32),
            pltpu.SMEM((_WAVE * 128,), jnp.float32),
            pltpu.SMEM((half,), jnp.float32),
            pltpu.SemaphoreType.DMA,
            pltpu.SemaphoreType.DMA,
        ],
    )
    def gather_kernel(t_hbm, x_hbm, out_hbm, idx_s, chunk_s, sel_s, sem,
                      gsem):
        cid = lax.axis_index("c")
        base = cid * half
        pltpu.async_copy(t_hbm.at[pl.ds(base, half)], idx_s, sem).wait()

        @pl.loop(0, half, step=_WAVE)
        def _(w):
            @pl.loop(0, _WAVE)
            def _(j):
                i = w + j
                t_al = pl.multiple_of((idx_s[i] >> 7) << 7, 128)
                pltpu.async_copy(
                    x_hbm.at[base + i].at[pl.ds(t_al, 128)],
                    chunk_s.at[pl.ds(j * 128, 128)], gsem)

            pltpu.make_async_copy(
                x_hbm.at[0].at[pl.ds(0, _WAVE * 128)], chunk_s, gsem).wait()

            @pl.loop(0, _WAVE)
            def _(j):
                i = w + j
                t = idx_s[i]
                sel_s[i] = chunk_s[j * 128 + (t & 127)]

        pltpu.async_copy(sel_s, out_hbm.at[pl.ds(base, half)], sem).wait()

    return gather_kernel(target, x)


def _rowsum_body(x_ref, s_ref, x0_ref, acc_ref):
    i = pl.program_id(0)
    nb = pl.num_programs(0)
    xb = x_ref[...]

    @pl.when(i == 0)
    def _init():
        acc_ref[...] = jnp.zeros_like(acc_ref)
        x0_ref[...] = xb[:, 0:1]

    @pl.when(i < nb - 1)
    def _full():
        acc_ref[...] += xb.reshape(_N, _CB // 128, 128).sum(axis=1)

    @pl.when(i == nb - 1)
    def _tail():
        col = i * _CB + lax.broadcasted_iota(jnp.int32, (_N, _CB), 1)
        xm = jnp.where(col < _SIZE, xb, 0.0)
        acc_ref[...] += xm.reshape(_N, _CB // 128, 128).sum(axis=1)
        s_ref[...] = jnp.sum(acc_ref[...], axis=1, keepdims=True)


def _tc_rowsum(x):
    nb = (_SIZE + _CB - 1) // _CB
    return pl.pallas_call(
        _rowsum_body,
        grid=(nb,),
        in_specs=[pl.BlockSpec((_N, _CB), lambda i: (0, i))],
        out_specs=[
            pl.BlockSpec((_N, 1), lambda i: (0, 0)),
            pl.BlockSpec((_N, 1), lambda i: (0, 0)),
        ],
        out_shape=[
            jax.ShapeDtypeStruct((_N, 1), jnp.float32),
            jax.ShapeDtypeStruct((_N, 1), jnp.float32),
        ],
        scratch_shapes=[pltpu.VMEM((_N, 128), jnp.float32)],
        compiler_params=pltpu.CompilerParams(
            dimension_semantics=("arbitrary",)),
    )(x)


def _combine_body(s_ref, x0_ref, g_ref, t_ref, out_ref):
    s = s_ref[...]
    g = g_ref[...]
    x0 = x0_ref[...]
    t = t_ref[...]
    row = _C - _EPS * s + _EPS * x0 + (_EPS - _CONF) * g
    row = jnp.where(t != _PAD, row, 0.0)
    out_ref[...] = jnp.sum(row, keepdims=True)


def _tc_combine(s, x0, g2, t2):
    out = pl.pallas_call(
        _combine_body,
        out_shape=jax.ShapeDtypeStruct((1, 1), jnp.float32),
    )(s, x0, g2, t2)
    return out[0, 0]


def kernel(x, target):
    n, size = x.shape
    g = _sc_gather(target, x)
    s, x0 = _tc_rowsum(x)
    return _tc_combine(s, x0, g.reshape(n, 1), target.reshape(n, 1))


# speedup vs baseline: 1.0183x; 1.0183x over previous
import dataclasses
import functools
import math

import jax
import jax.numpy as jnp
from jax import lax
from jax.experimental import pallas as pl
from jax.experimental.pallas import tpu as pltpu
from jax.experimental.pallas import tpu_sc as plsc

_SIZE = 100000
_PAD = 0
_SMOOTHING = 0.1
_CONF = 1.0 - _SMOOTHING
_EPS = _SMOOTHING / (_SIZE - 2)
_C = (_SIZE - 2) * _EPS * math.log(_EPS) + _CONF * math.log(_CONF)

_N = 1024
_CB = 4096
_CT = 81920
_SLAB = _SIZE - _CT
_SLAB_RD = ((_SLAB + 127) // 128) * 128
_WAVE = 64
_NWORK = 32
_RPW = _N // _NWORK
_CHUNK = 2048


def _sc_gather(target, x):
    mesh = plsc.ScalarSubcoreMesh(axis_name="c", num_cores=2)
    half = _N // 2

    @functools.partial(
        pl.kernel,
        out_type=jax.ShapeDtypeStruct((_N,), jnp.float32),
        mesh=mesh,
        scratch_types=[
            pltpu.SMEM((half,), jnp.int32),
            pltpu.SMEM((_WAVE * 128,), jnp.float32),
            pltpu.SMEM((half,), jnp.float32),
            pltpu.SemaphoreType.DMA,
            pltpu.SemaphoreType.DMA,
        ],
    )
    def gather_kernel(t_hbm, x_hbm, out_hbm, idx_s, chunk_s, sel_s, sem,
                      gsem):
        cid = lax.axis_index("c")
        base = cid * half
        pltpu.async_copy(t_hbm.at[pl.ds(base, half)], idx_s, sem).wait()

        @pl.loop(0, half, step=_WAVE)
        def _(w):
            @pl.loop(0, _WAVE)
            def _(j):
                i = w + j
                t_al = pl.multiple_of((idx_s[i] >> 7) << 7, 128)
                pltpu.async_copy(
                    x_hbm.at[base + i].at[pl.ds(t_al, 128)],
                    chunk_s.at[pl.ds(j * 128, 128)], gsem)

            pltpu.make_async_copy(
                x_hbm.at[0].at[pl.ds(0, _WAVE * 128)], chunk_s, gsem).wait()

            @pl.loop(0, _WAVE)
            def _(j):
                i = w + j
                t = idx_s[i]
                sel_s[i] = chunk_s[j * 128 + (t & 127)]

        pltpu.async_copy(sel_s, out_hbm.at[pl.ds(base, half)], sem).wait()

    return gather_kernel(target, x)


def _sc_slabsum(x):
    mesh = plsc.VectorSubcoreMesh(core_axis_name="c", subcore_axis_name="s")
    nfull = _SLAB_RD // _CHUNK
    tail = _SLAB_RD - nfull * _CHUNK

    @functools.partial(
        pl.kernel,
        out_type=jax.ShapeDtypeStruct((_N, 16), jnp.float32),
        mesh=mesh,
        scratch_types=[
            pltpu.VMEM((2, _SLAB_RD), jnp.float32),
            pltpu.VMEM((_RPW, 16), jnp.float32),
            pltpu.SemaphoreType.DMA,
            pltpu.SemaphoreType.DMA,
        ],
    )
    def slab_kernel(x_hbm, out_hbm, bufs, out_v, sem0, sem1):
        wid = lax.axis_index("s") * 2 + lax.axis_index("c")
        base = wid * _RPW
        sems = (sem0, sem1)

        def fire(j, slot):
            row = x_hbm.at[base + j]
            for c in range(nfull):
                pltpu.async_copy(
                    row.at[pl.ds(_CT + c * _CHUNK, _CHUNK)],
                    bufs.at[slot].at[pl.ds(c * _CHUNK, _CHUNK)],
                    sems[slot])
            if tail:
                toff = pl.multiple_of(
                    jnp.asarray(_CT + nfull * _CHUNK, jnp.int32), 128)
                pltpu.async_copy(
                    row.at[pl.ds(toff, tail)],
                    bufs.at[slot].at[pl.ds(nfull * _CHUNK, tail)],
                    sems[slot])

        def drain(slot):
            pltpu.make_async_copy(
                x_hbm.at[0].at[pl.ds(0, _SLAB_RD)], bufs.at[slot],
                sems[slot]).wait()

        def accum(j, slot):
            row_buf = bufs.at[slot]

            def add16(i, a):
                off = pl.multiple_of(i * 16, 16)
                return a + row_buf[pl.ds(off, 16)]

            acc = lax.fori_loop(0, _SLAB // 16, add16,
                                jnp.zeros((16,), jnp.float32), unroll=8)
            out_v.at[j][...] = acc

        fire(0, 0)

        @pl.loop(0, _RPW, step=2)
        def _(j0):
            fire(j0 + 1, 1)
            drain(0)
            accum(j0, 0)

            @pl.when(j0 + 2 < _RPW)
            def _():
                fire(j0 + 2, 0)

            drain(1)
            accum(j0 + 1, 1)

        pltpu.sync_copy(out_v, out_hbm.at[pl.ds(base, _RPW)])

    return slab_kernel(x)


def _rowsum_body(x_ref, s_ref, x0_ref, acc_ref):
    i = pl.program_id(0)
    nb = pl.num_programs(0)
    xb = x_ref[...]

    @pl.when(i == 0)
    def _init():
        acc_ref[...] = jnp.zeros_like(acc_ref)
        x0_ref[...] = xb[:, 0:1]

    acc_ref[...] += xb.reshape(_N, _CB // 128, 128).sum(axis=1)

    @pl.when(i == nb - 1)
    def _finish():
        s_ref[...] = jnp.sum(acc_ref[...], axis=1, keepdims=True)


def _tc_rowsum(x):
    nb = _CT // _CB
    return pl.pallas_call(
        _rowsum_body,
        grid=(nb,),
        in_specs=[pl.BlockSpec((_N, _CB), lambda i: (0, i))],
        out_specs=[
            pl.BlockSpec((_N, 1), lambda i: (0, 0)),
            pl.BlockSpec((_N, 1), lambda i: (0, 0)),
        ],
        out_shape=[
            jax.ShapeDtypeStruct((_N, 1), jnp.float32),
            jax.ShapeDtypeStruct((_N, 1), jnp.float32),
        ],
        scratch_shapes=[pltpu.VMEM((_N, 128), jnp.float32)],
        compiler_params=pltpu.CompilerParams(
            dimension_semantics=("arbitrary",)),
    )(x)


def _combine_body(s_ref, slab_ref, x0_ref, g_ref, t_ref, out_ref):
    s = s_ref[...] + jnp.sum(slab_ref[...], axis=1, keepdims=True)
    g = g_ref[...]
    x0 = x0_ref[...]
    t = t_ref[...]
    row = _C - _EPS * s + _EPS * x0 + (_EPS - _CONF) * g
    row = jnp.where(t != _PAD, row, 0.0)
    out_ref[...] = jnp.sum(row, keepdims=True)


def _tc_combine(s, slab, x0, g2, t2):
    out = pl.pallas_call(
        _combine_body,
        out_shape=jax.ShapeDtypeStruct((1, 1), jnp.float32),
    )(s, slab, x0, g2, t2)
    return out[0, 0]


def kernel(x, target):
    n, size = x.shape
    g = _sc_gather(target, x)
    slab = _sc_slabsum(x)
    s, x0 = _tc_rowsum(x)
    return _tc_combine(s, slab, x0, g.reshape(n, 1), target.reshape(n, 1))
